# Initial kernel scaffold; baseline (speedup 1.0000x reference)
import functools

import jax
import jax.numpy as jnp
from jax import lax
from jax.experimental import pallas as pl
from jax.experimental.pallas import tpu as pltpu
from jax.experimental.pallas import tpu_sc as plsc

OUT_DIM = 16384
IN_DIM = 16384
NNZ = 1048576
B = 64
BIAS_NNZ = 16384

NC = 2
NS = 16
LANES = 16
BH = B // NC
CH = 512
CHB = CH // 128
PER_TILE = NNZ // NS
N_OUTER = PER_TILE // CH
ROWS_PER_TILE = OUT_DIM // NS


def _sc_body(in_hbm, rows_hbm, cols_hbm, vals_hbm, bias_hbm, out_hbm,
             out_sh, rows_v, cols_v, vals_v, gath_v):
    c = lax.axis_index("c")
    s = lax.axis_index("s")

    for p in range(ROWS_PER_TILE // CH):
        rbase = s * ROWS_PER_TILE + p * CH

        pltpu.sync_copy(bias_hbm.at[pl.ds(rbase, CH)], vals_v)

        @pl.loop(0, CH)
        def _bias(k):
            bv = vals_v[k]
            gath_v[k, pl.ds(0, LANES)] = jnp.full((LANES,), bv, jnp.float32)
            gath_v[k, pl.ds(LANES, LANES)] = jnp.full((LANES,), bv, jnp.float32)

        pltpu.sync_copy(gath_v, out_sh.at[pl.ds(rbase, CH)])

    plsc.subcore_barrier()

    col_off = c * IN_DIM

    @pl.loop(0, N_OUTER)
    def _outer(i):
        nbase = s * PER_TILE + i * CH
        idx_base = s * (PER_TILE // 128) + i * CHB

        pltpu.sync_copy(rows_hbm.at[pl.ds(idx_base, CHB)], rows_v)
        pltpu.sync_copy(cols_hbm.at[pl.ds(idx_base, CHB)], cols_v)
        pltpu.sync_copy(vals_hbm.at[pl.ds(nbase, CH)], vals_v)

        for qr in range(CHB):
            @pl.loop(0, 128 // LANES)
            def _off(q):
                cols_v[qr, pl.ds(q * LANES, LANES)] = (
                    cols_v[qr, pl.ds(q * LANES, LANES)] + col_off)

        for q in range(CHB):
            pltpu.sync_copy(in_hbm.at[cols_v.at[q]],
                            gath_v.at[pl.ds(q * 128, 128)])

        @pl.loop(0, CH)
        def _scale(k):
            v = vals_v[k]
            gath_v[k, pl.ds(0, LANES)] = gath_v[k, pl.ds(0, LANES)] * v
            gath_v[k, pl.ds(LANES, LANES)] = gath_v[k, pl.ds(LANES, LANES)] * v

        for q in range(CHB):
            pltpu.sync_copy(gath_v.at[pl.ds(q * 128, 128)],
                            out_sh.at[rows_v.at[q]], add=True)

    plsc.subcore_barrier()

    pltpu.sync_copy(out_sh.at[pl.ds(s * ROWS_PER_TILE, ROWS_PER_TILE)],
                    out_hbm.at[c, pl.ds(s * ROWS_PER_TILE, ROWS_PER_TILE)])


@jax.jit
def _run(in_parts, rows2d, cols2d, wvals, bias_vals):
    mesh = plsc.VectorSubcoreMesh(
        core_axis_name="c", subcore_axis_name="s",
        num_cores=NC, num_subcores=NS)
    f = pl.kernel(
        _sc_body,
        out_type=jax.ShapeDtypeStruct((NC, OUT_DIM, BH), jnp.float32),
        mesh=mesh,
        scratch_types=[
            pltpu.VMEM_SHARED((OUT_DIM, BH), jnp.float32),
            pltpu.VMEM((CHB, 128), jnp.int32),
            pltpu.VMEM((CHB, 128), jnp.int32),
            pltpu.VMEM((CH,), jnp.float32),
            pltpu.VMEM((CH, BH), jnp.float32),
        ],
    )
    return f(in_parts, rows2d, cols2d, wvals, bias_vals)


def kernel(input, weight_indices, weight_values, bias_indices, bias_values):
    in_parts = (input.T.reshape(IN_DIM, NC, BH)
                .transpose(1, 0, 2).reshape(NC * IN_DIM, BH))
    rows2d = weight_indices[0].astype(jnp.int32).reshape(NNZ // 128, 128)
    cols2d = weight_indices[1].astype(jnp.int32).reshape(NNZ // 128, 128)
    out3 = _run(in_parts, rows2d, cols2d,
                weight_values.astype(jnp.float32),
                bias_values.astype(jnp.float32))
    return out3.transpose(0, 2, 1).reshape(B, OUT_DIM)


# speedup vs baseline: 8.1198x; 8.1198x over previous
import functools

import jax
import jax.numpy as jnp
from jax import lax
from jax.experimental import pallas as pl
from jax.experimental.pallas import tpu as pltpu
from jax.experimental.pallas import tpu_sc as plsc

OUT_DIM = 16384
IN_DIM = 16384
NNZ = 1048576
B = 64
BIAS_NNZ = 16384

NC = 2
NS = 16
LANES = 16
BH = B // NC
CH = 512
CHB = CH // 128
PER_TILE = NNZ // NS
N_OUTER = PER_TILE // CH
ROWS_PER_TILE = OUT_DIM // NS


def _sc_body(in_hbm, rows_hbm, cols_hbm, vals_hbm, bias_hbm, out_hbm,
             out_sh, rows_v, cols_v, vals_v, gath_v):
    c = lax.axis_index("c")
    s = lax.axis_index("s")

    for p in range(ROWS_PER_TILE // CH):
        rbase = s * ROWS_PER_TILE + p * CH

        pltpu.sync_copy(bias_hbm.at[pl.ds(rbase, CH)], vals_v)

        @pl.loop(0, CH // LANES)
        def _bias(t):
            bv16 = vals_v[pl.ds(t * LANES, LANES)]
            for l in range(LANES):
                k = t * LANES + l
                row = jnp.full((LANES,), bv16[l], jnp.float32)
                gath_v[k, pl.ds(0, LANES)] = row
                gath_v[k, pl.ds(LANES, LANES)] = row

        pltpu.sync_copy(gath_v, out_sh.at[pl.ds(rbase, CH)])

    plsc.subcore_barrier()

    col_off = c * IN_DIM

    @pl.loop(0, N_OUTER)
    def _outer(i):
        nbase = s * PER_TILE + i * CH
        idx_base = s * (PER_TILE // 128) + i * CHB

        pltpu.sync_copy(rows_hbm.at[pl.ds(idx_base, CHB)], rows_v)
        pltpu.sync_copy(cols_hbm.at[pl.ds(idx_base, CHB)], cols_v)
        pltpu.sync_copy(vals_hbm.at[pl.ds(nbase, CH)], vals_v)

        for qr in range(CHB):
            @pl.loop(0, 128 // LANES)
            def _off(q):
                cols_v[qr, pl.ds(q * LANES, LANES)] = (
                    cols_v[qr, pl.ds(q * LANES, LANES)] + col_off)

        for q in range(CHB):
            pltpu.sync_copy(in_hbm.at[cols_v.at[q]],
                            gath_v.at[pl.ds(q * 128, 128)])

        @pl.loop(0, CH // LANES)
        def _scale(t):
            v16 = vals_v[pl.ds(t * LANES, LANES)]
            for l in range(LANES):
                k = t * LANES + l
                v = v16[l]
                gath_v[k, pl.ds(0, LANES)] = gath_v[k, pl.ds(0, LANES)] * v
                gath_v[k, pl.ds(LANES, LANES)] = (
                    gath_v[k, pl.ds(LANES, LANES)] * v)

        for q in range(CHB):
            pltpu.sync_copy(gath_v.at[pl.ds(q * 128, 128)],
                            out_sh.at[rows_v.at[q]], add=True)

    plsc.subcore_barrier()

    pltpu.sync_copy(out_sh.at[pl.ds(s * ROWS_PER_TILE, ROWS_PER_TILE)],
                    out_hbm.at[c, pl.ds(s * ROWS_PER_TILE, ROWS_PER_TILE)])


@jax.jit
def _run(in_parts, rows2d, cols2d, wvals, bias_vals):
    mesh = plsc.VectorSubcoreMesh(
        core_axis_name="c", subcore_axis_name="s",
        num_cores=NC, num_subcores=NS)
    f = pl.kernel(
        _sc_body,
        out_type=jax.ShapeDtypeStruct((NC, OUT_DIM, BH), jnp.float32),
        mesh=mesh,
        compiler_params=pltpu.CompilerParams(use_tc_tiling_on_sc=False),
        scratch_types=[
            pltpu.VMEM_SHARED((OUT_DIM, BH), jnp.float32),
            pltpu.VMEM((CHB, 128), jnp.int32),
            pltpu.VMEM((CHB, 128), jnp.int32),
            pltpu.VMEM((CH,), jnp.float32),
            pltpu.VMEM((CH, BH), jnp.float32),
        ],
    )
    return f(in_parts, rows2d, cols2d, wvals, bias_vals)


def kernel(input, weight_indices, weight_values, bias_indices, bias_values):
    in_parts = (input.T.reshape(IN_DIM, NC, BH)
                .transpose(1, 0, 2).reshape(NC * IN_DIM, BH))
    rows2d = weight_indices[0].astype(jnp.int32).reshape(NNZ // 128, 128)
    cols2d = weight_indices[1].astype(jnp.int32).reshape(NNZ // 128, 128)
    out3 = _run(in_parts, rows2d, cols2d,
                weight_values.astype(jnp.float32),
                bias_values.astype(jnp.float32))
    return out3.transpose(0, 2, 1).reshape(B, OUT_DIM)


# speedup vs baseline: 18.9969x; 2.3396x over previous
import functools

import jax
import jax.numpy as jnp
from jax import lax
from jax.experimental import pallas as pl
from jax.experimental.pallas import tpu as pltpu
from jax.experimental.pallas import tpu_sc as plsc

OUT_DIM = 16384
IN_DIM = 16384
NNZ = 1048576
B = 64
BIAS_NNZ = 16384

NC = 2
NS = 16
LANES = 16
BH = B // NC
CH = 512
CHB = CH // 128
PER_TILE = NNZ // NS
N_OUTER = PER_TILE // CH
ROWS_PER_TILE = OUT_DIM // NS


def _sc_body(in_hbm, rows_hbm, cols_hbm, vals_hbm, bias_hbm, out_hbm,
             out_sh, rows_v, cols_v, vals_v, gath_v,
             sem_i0, sem_i1, sem_g0, sem_g1):
    c = lax.axis_index("c")
    s = lax.axis_index("s")
    sem_i = (sem_i0, sem_i1)
    sem_g = (sem_g0, sem_g1)

    for p in range(ROWS_PER_TILE // CH):
        rbase = s * ROWS_PER_TILE + p * CH

        pltpu.sync_copy(bias_hbm.at[pl.ds(rbase, CH)], vals_v.at[0])

        @pl.loop(0, CH // LANES)
        def _bias(t):
            bv16 = vals_v[0, pl.ds(t * LANES, LANES)]
            for l in range(LANES):
                k = t * LANES + l
                row = jnp.full((LANES,), bv16[l], jnp.float32)
                gath_v[0, k, pl.ds(0, LANES)] = row
                gath_v[0, k, pl.ds(LANES, LANES)] = row

        pltpu.sync_copy(gath_v.at[0], out_sh.at[pl.ds(rbase, CH)])

    plsc.subcore_barrier()

    col_off = c * IN_DIM

    def idx_issue(i, b):
        idx_base = s * (PER_TILE // 128) + i * CHB
        nbase = s * PER_TILE + i * CH
        pltpu.async_copy(rows_hbm.at[pl.ds(idx_base, CHB)], rows_v.at[b],
                         sem_i[b])
        pltpu.async_copy(cols_hbm.at[pl.ds(idx_base, CHB)], cols_v.at[b],
                         sem_i[b])
        pltpu.async_copy(vals_hbm.at[pl.ds(nbase, CH)], vals_v.at[b],
                         sem_i[b])

    def idx_wait(b):
        pltpu.make_async_copy(rows_hbm.at[pl.ds(0, CHB)], rows_v.at[b],
                              sem_i[b]).wait()
        pltpu.make_async_copy(cols_hbm.at[pl.ds(0, CHB)], cols_v.at[b],
                              sem_i[b]).wait()
        pltpu.make_async_copy(vals_hbm.at[pl.ds(0, CH)], vals_v.at[b],
                              sem_i[b]).wait()

    def gather_issue(b):
        for qr in range(CHB):
            @pl.loop(0, 128 // LANES)
            def _off(q):
                cols_v[b, qr, pl.ds(q * LANES, LANES)] = (
                    cols_v[b, qr, pl.ds(q * LANES, LANES)] + col_off)
        for q in range(CHB):
            pltpu.async_copy(in_hbm.at[cols_v.at[b, q]],
                             gath_v.at[b, pl.ds(q * 128, 128)], sem_g[b])

    def gather_wait(b):
        for q in range(CHB):
            pltpu.make_async_copy(in_hbm.at[cols_v.at[b, q]],
                                  gath_v.at[b, pl.ds(q * 128, 128)],
                                  sem_g[b]).wait()

    def scale(b):
        @pl.loop(0, CH // LANES)
        def _scale(t):
            v16 = vals_v[b, pl.ds(t * LANES, LANES)]
            for l in range(LANES):
                k = t * LANES + l
                v = v16[l]
                gath_v[b, k, pl.ds(0, LANES)] = (
                    gath_v[b, k, pl.ds(0, LANES)] * v)
                gath_v[b, k, pl.ds(LANES, LANES)] = (
                    gath_v[b, k, pl.ds(LANES, LANES)] * v)

    def scatter(b):
        for q in range(CHB):
            pltpu.sync_copy(gath_v.at[b, pl.ds(q * 128, 128)],
                            out_sh.at[rows_v.at[b, q]], add=True)

    def steady(i, b):
        ob = 1 - b
        gather_wait(b)
        idx_wait(ob)
        gather_issue(ob)
        scale(b)
        scatter(b)

        @pl.when(i + 2 < N_OUTER)
        def _():
            idx_issue(i + 2, b)

    idx_issue(0, 0)
    idx_issue(1, 1)
    idx_wait(0)
    gather_issue(0)

    @pl.loop(0, (N_OUTER - 2) // 2)
    def _outer(ii):
        for b in range(2):
            steady(ii * 2 + b, b)

    gather_wait(0)
    idx_wait(1)
    gather_issue(1)
    scale(0)
    scatter(0)
    gather_wait(1)
    scale(1)
    scatter(1)

    plsc.subcore_barrier()

    pltpu.sync_copy(out_sh.at[pl.ds(s * ROWS_PER_TILE, ROWS_PER_TILE)],
                    out_hbm.at[c, pl.ds(s * ROWS_PER_TILE, ROWS_PER_TILE)])


@jax.jit
def _run(in_parts, rows2d, cols2d, wvals, bias_vals):
    mesh = plsc.VectorSubcoreMesh(
        core_axis_name="c", subcore_axis_name="s",
        num_cores=NC, num_subcores=NS)
    f = pl.kernel(
        _sc_body,
        out_type=jax.ShapeDtypeStruct((NC, OUT_DIM, BH), jnp.float32),
        mesh=mesh,
        compiler_params=pltpu.CompilerParams(use_tc_tiling_on_sc=False),
        scratch_types=[
            pltpu.VMEM_SHARED((OUT_DIM, BH), jnp.float32),
            pltpu.VMEM((2, CHB, 128), jnp.int32),
            pltpu.VMEM((2, CHB, 128), jnp.int32),
            pltpu.VMEM((2, CH), jnp.float32),
            pltpu.VMEM((2, CH, BH), jnp.float32),
            pltpu.SemaphoreType.DMA,
            pltpu.SemaphoreType.DMA,
            pltpu.SemaphoreType.DMA,
            pltpu.SemaphoreType.DMA,
        ],
    )
    return f(in_parts, rows2d, cols2d, wvals, bias_vals)


def kernel(input, weight_indices, weight_values, bias_indices, bias_values):
    in_parts = (input.T.reshape(IN_DIM, NC, BH)
                .transpose(1, 0, 2).reshape(NC * IN_DIM, BH))
    rows2d = weight_indices[0].astype(jnp.int32).reshape(NNZ // 128, 128)
    cols2d = weight_indices[1].astype(jnp.int32).reshape(NNZ // 128, 128)
    out3 = _run(in_parts, rows2d, cols2d,
                weight_values.astype(jnp.float32),
                bias_values.astype(jnp.float32))
    return out3.transpose(0, 2, 1).reshape(B, OUT_DIM)
